# baseline (speedup 1.0000x reference)
import dataclasses
import functools

import jax
import jax.numpy as jnp
from jax import lax
from jax.experimental import pallas as pl
from jax.experimental.pallas import tpu as pltpu
from jax.experimental.pallas import tpu_sc as plsc

NUM_OUTPUTS = 128
LANES = 16
NVREG = NUM_OUTPUTS // LANES
NUM_CORES = 2
NUM_SUBCORES = 16
NW = NUM_CORES * NUM_SUBCORES

CHUNK = 16
PADL = 128


def _make_kernel(batch, max_active):
    rows_per_w = batch // NW
    nchunk = rows_per_w // CHUNK
    rows_per_chunk = CHUNK * max_active
    assert batch % (NW * CHUNK) == 0
    assert nchunk % 2 == 0
    assert max_active <= PADL

    mesh = plsc.VectorSubcoreMesh(core_axis_name="c", subcore_axis_name="s")
    out_sds = jax.ShapeDtypeStruct((batch, NUM_OUTPUTS), jnp.float32)
    idx_buf = pltpu.VMEM((CHUNK, PADL), jnp.int32)
    vals_buf = pltpu.VMEM((CHUNK, PADL), jnp.float32)
    row_buf = pltpu.VMEM((rows_per_chunk, NUM_OUTPUTS), jnp.float32)
    out_buf = pltpu.VMEM((CHUNK, NUM_OUTPUTS), jnp.float32)

    cp = pltpu.CompilerParams()
    if "needs_layout_passes" in pltpu.CompilerParams.__dataclass_fields__:
        cp = dataclasses.replace(cp, needs_layout_passes=False)

    @functools.partial(
        pl.kernel,
        out_type=out_sds,
        mesh=mesh,
        compiler_params=cp,
        scratch_types=[
            idx_buf, idx_buf,
            vals_buf, vals_buf,
            row_buf, row_buf,
            out_buf, out_buf,
            pltpu.VMEM((NUM_OUTPUTS,), jnp.float32),
            pltpu.SemaphoreType.DMA,
            pltpu.SemaphoreType.DMA,
            pltpu.SemaphoreType.DMA,
            pltpu.SemaphoreType.DMA,
            pltpu.SemaphoreType.DMA,
            pltpu.SemaphoreType.DMA,
            pltpu.SemaphoreType.DMA,
            pltpu.SemaphoreType.DMA,
        ],
    )
    def k(idx_hbm, vals_hbm, w_hbm, bias_hbm, out_hbm,
          idx_a, idx_b, vals_a, vals_b, rows_a, rows_b, out_a, out_b,
          bias_v, sem_ia, sem_ib, sem_va, sem_vb,
          sem_ga, sem_gb, sem_oa, sem_ob):
        wid = lax.axis_index("s") * NUM_CORES + lax.axis_index("c")
        base_row = wid * rows_per_w
        pltpu.sync_copy(bias_hbm, bias_v)

        def fire_idx(c, idx_v, sem):
            pltpu.async_copy(
                idx_hbm.at[pl.ds(base_row + c * CHUNK, CHUNK)], idx_v, sem)

        def drain_idx(idx_v, sem):
            pltpu.make_async_copy(
                idx_hbm.at[pl.ds(0, CHUNK)], idx_v, sem).wait()

        def fire_vals(c, vals_v, sem):
            pltpu.async_copy(
                vals_hbm.at[pl.ds(base_row + c * CHUNK, CHUNK)], vals_v, sem)

        def drain_vals(vals_v, sem):
            pltpu.make_async_copy(
                vals_hbm.at[pl.ds(0, CHUNK)], vals_v, sem).wait()

        def fire_gather(idx_v, rows_v, sem):
            for rr in range(CHUNK):
                pltpu.async_copy(
                    w_hbm.at[idx_v.at[rr, pl.ds(0, max_active)]],
                    rows_v.at[pl.ds(rr * max_active, max_active)],
                    sem,
                )

        def drain_rows(rows_v, sem):
            pltpu.make_async_copy(
                w_hbm.at[pl.ds(0, rows_per_chunk)], rows_v, sem).wait()

        def drain_out(out_v, sem):
            pltpu.make_async_copy(out_hbm.at[pl.ds(0, CHUNK)], out_v, sem).wait()

        def compute(vals_v, rows_v, out_v, c, sem):
            bias_r = [bias_v[pl.ds(kk * LANES, LANES)] for kk in range(NVREG)]

            @pl.loop(0, CHUNK)
            def _(r):
                acc = list(bias_r)
                rbase = r * max_active
                v0 = vals_v[r, pl.ds(0, LANES)]
                v1 = vals_v[r, pl.ds(LANES, LANES)]
                for j in range(max_active):
                    s = v0[j] if j < LANES else v1[j - LANES]
                    v = jnp.broadcast_to(s, (LANES,))
                    for kk in range(NVREG):
                        acc[kk] = acc[kk] + v * rows_v[rbase + j,
                                                       pl.ds(kk * LANES, LANES)]
                for kk in range(NVREG):
                    out_v[r, pl.ds(kk * LANES, LANES)] = acc[kk]

            pltpu.async_copy(
                out_v,
                out_hbm.at[pl.ds(base_row + c * CHUNK, CHUNK)],
                sem)

        fire_idx(0, idx_a, sem_ia)
        fire_vals(0, vals_a, sem_va)
        fire_vals(1, vals_b, sem_vb)
        drain_idx(idx_a, sem_ia)
        fire_gather(idx_a, rows_a, sem_ga)
        fire_idx(1, idx_b, sem_ib)

        @pl.loop(0, nchunk, step=2)
        def _(c):
            drain_idx(idx_b, sem_ib)
            fire_gather(idx_b, rows_b, sem_gb)

            drain_rows(rows_a, sem_ga)

            @pl.when(c + 2 < nchunk)
            def _():
                fire_idx(c + 2, idx_a, sem_ia)

            @pl.when(c > 0)
            def _():
                drain_out(out_a, sem_oa)
            drain_vals(vals_a, sem_va)
            compute(vals_a, rows_a, out_a, c, sem_oa)

            @pl.when(c + 2 < nchunk)
            def _():
                fire_vals(c + 2, vals_a, sem_va)
                drain_idx(idx_a, sem_ia)
                fire_gather(idx_a, rows_a, sem_ga)

            drain_rows(rows_b, sem_gb)

            @pl.when(c + 3 < nchunk)
            def _():
                fire_idx(c + 3, idx_b, sem_ib)

            @pl.when(c > 0)
            def _():
                drain_out(out_b, sem_ob)
            drain_vals(vals_b, sem_vb)
            compute(vals_b, rows_b, out_b, c + 1, sem_ob)

            @pl.when(c + 3 < nchunk)
            def _():
                fire_vals(c + 3, vals_b, sem_vb)

        drain_out(out_a, sem_oa)
        drain_out(out_b, sem_ob)

    return k


def kernel(feature_indices_0, feature_values_0, feature_indices_1,
           feature_values_1, weight, bias):
    batch, max_active = feature_indices_0.shape
    padw = ((0, 0), (0, PADL - max_active))
    k = _make_kernel(batch, max_active)
    out0 = k(jnp.pad(feature_indices_0, padw),
             jnp.pad(feature_values_0, padw), weight, bias)
    out1 = k(jnp.pad(feature_indices_1, padw),
             jnp.pad(feature_values_1, padw), weight, bias)
    return (out0, out1)
